# baseline (speedup 1.0000x reference)
import jax
import jax.numpy as jnp
from jax.experimental import pallas as pl
from jax.experimental.pallas import tpu as pltpu

S = 5
BM = 80


def _gcn_body(*refs):
    a_refs = refs[:S]
    x_ref, deg_ref, wt_ref, b_ref, out_ref = refs[S:]
    i = pl.program_id(0)
    xb = x_ref[...].astype(jnp.bfloat16)
    accs = [jnp.dot(a_ref[...].astype(jnp.bfloat16), xb,
                    preferred_element_type=jnp.float32) for a_ref in a_refs]
    acc = jnp.concatenate(accs, axis=0)
    xr = x_ref[pl.ds(i * (S * BM), S * BM), :]
    inv = 1.0 / deg_ref[...]
    pool = inv * (acc + xr) + xr
    out = jnp.dot(pool, wt_ref[...], preferred_element_type=jnp.float32)
    out_ref[...] = jnp.maximum(out + b_ref[...], 0.0)


def _strip_spec(n, j):
    return pl.BlockSpec((BM, n), lambda i, j=j: (S * i + j, 0))


@jax.jit
def kernel(input_tensor, adjacency_matrix, node_degree, W, b):
    n, d_in = input_tensor.shape
    d_out = W.shape[0]
    wt = W.T
    b2 = b.reshape(1, d_out)

    return pl.pallas_call(
        _gcn_body,
        grid=(n // (S * BM),),
        in_specs=[_strip_spec(n, j) for j in range(S)] + [
            pl.BlockSpec((n, d_in), lambda i: (0, 0)),
            pl.BlockSpec((S * BM, 1), lambda i: (i, 0)),
            pl.BlockSpec((d_in, d_out), lambda i: (0, 0)),
            pl.BlockSpec((1, d_out), lambda i: (0, 0)),
        ],
        out_specs=pl.BlockSpec((S * BM, d_out), lambda i: (i, 0)),
        out_shape=jax.ShapeDtypeStruct((n, d_out), jnp.float32),
        compiler_params=pltpu.CompilerParams(
            dimension_semantics=("parallel",)),
    )(*([adjacency_matrix] * S), input_tensor, node_degree, wt, b2)
